# baseline (speedup 1.0000x reference)
import functools

import jax
import jax.numpy as jnp
from jax import lax
from jax.experimental import pallas as pl
from jax.experimental.pallas import tpu as pltpu
from jax.experimental.pallas import tpu_sc as plsc

_K = 8192
_D = 256
_B = 32768
_BR = 128
_NB = _B // _BR
_CC = 0.25


def _vq_body(x_ref, e_ref, idx_ref, enc_ref, cnt_ref, loss_ref, bsq_ref):
    i = pl.program_id(0)

    @pl.when(i == 0)
    def _():
        e0 = e_ref[...]
        bsq_ref[...] = jnp.sum(e0 * e0, axis=1).reshape(1, _K)

    x = x_ref[...]
    a = jnp.sum(x * x, axis=1, keepdims=True)
    b = bsq_ref[...]
    c = jax.lax.dot_general(
        x, e_ref[...], (((1,), (1,)), ((), ())),
        preferred_element_type=jnp.float32)
    d = (a + b) - 2.0 * c
    dmin = jnp.min(d, axis=1, keepdims=True)
    col = jax.lax.broadcasted_iota(jnp.int32, (_BR, _K), 1)
    idx = jnp.min(jnp.where(d == dmin, col, _K), axis=1).astype(jnp.int32)
    idx_ref[0, 0, :] = idx
    onehot = (col == idx[:, None]).astype(jnp.float32)
    enc_ref[...] = onehot
    ones_row = jnp.ones((1, _BR), dtype=jnp.float32)
    pcnt = jax.lax.dot_general(
        ones_row, onehot, (((1,), (0,)), ((), ())),
        preferred_element_type=jnp.float32)
    ploss = jnp.sum(dmin.reshape(2, _BR // 2), axis=0, keepdims=True)

    @pl.when(i == 0)
    def _():
        cnt_ref[...] = pcnt
        loss_ref[...] = ploss

    @pl.when(i > 0)
    def _():
        cnt_ref[...] += pcnt
        loss_ref[...] += ploss


_vq_call = pl.pallas_call(
    _vq_body,
    grid=(_NB,),
    in_specs=[
        pl.BlockSpec((_BR, _D), lambda i: (i, 0)),
        pl.BlockSpec((_K, _D), lambda i: (0, 0)),
    ],
    out_specs=[
        pl.BlockSpec((1, 1, _BR), lambda i: (i, 0, 0)),
        pl.BlockSpec((_BR, _K), lambda i: (i, 0)),
        pl.BlockSpec((1, _K), lambda i: (0, 0)),
        pl.BlockSpec((1, _BR // 2), lambda i: (0, 0)),
    ],
    out_shape=[
        jax.ShapeDtypeStruct((_NB, 1, _BR), jnp.int32),
        jax.ShapeDtypeStruct((_B, _K), jnp.float32),
        jax.ShapeDtypeStruct((1, _K), jnp.float32),
        jax.ShapeDtypeStruct((1, _BR // 2), jnp.float32),
    ],
    scratch_shapes=[pltpu.VMEM((1, _K), jnp.float32)],
)

_SC_INFO = plsc.get_sparse_core_info()
_NC = _SC_INFO.num_cores
_NS = _SC_INFO.num_subcores
_NW = _NC * _NS
_BPW = _B // _NW
_CH = 128
_NCH = _BPW // _CH


@functools.partial(
    pl.kernel,
    mesh=plsc.VectorSubcoreMesh(core_axis_name="c", subcore_axis_name="s"),
    out_type=jax.ShapeDtypeStruct((_B, _D), jnp.float32),
    scratch_types=[
        pltpu.VMEM((_CH,), jnp.int32),
        pltpu.VMEM((_CH, _D), jnp.float32),
        pltpu.SemaphoreType.DMA,
    ],
)
def _sc_gather(idx_hbm, table_hbm, out_hbm, idx_v, rows_v, sem):
    wid = lax.axis_index("s") * _NC + lax.axis_index("c")
    base = wid * _BPW
    for ci in range(_NCH):
        off = base + ci * _CH
        pltpu.sync_copy(idx_hbm.at[pl.ds(off, _CH)], idx_v)
        pltpu.async_copy(table_hbm.at[idx_v], rows_v, sem).wait()
        pltpu.sync_copy(rows_v, out_hbm.at[pl.ds(off, _CH)])


def kernel(inputs, label, embedding_weight):
    idx3, enc, cnt, losspart = _vq_call(inputs, embedding_weight)
    q = _sc_gather(idx3.reshape(_B), embedding_weight)
    a = jnp.sum(losspart) / (_B * _D)
    loss = a + _CC * a
    p = cnt[0] / _B
    perplexity = jnp.exp(-jnp.sum(p * jnp.log(p + 1e-10)))
    return (loss, q, perplexity, enc)


# speedup vs baseline: 1.5755x; 1.5755x over previous
import functools

import jax
import jax.numpy as jnp
from jax import lax
from jax.experimental import pallas as pl
from jax.experimental.pallas import tpu as pltpu
from jax.experimental.pallas import tpu_sc as plsc

_K = 8192
_D = 256
_B = 32768
_BR = 256
_NB = _B // _BR
_CC = 0.25


def _vq_body(x_ref, e_ref, idx_ref, enc_ref, cnt_ref, loss_ref, bsq_ref,
             cnt_acc, loss_acc):
    i = pl.program_id(0)

    @pl.when(i == 0)
    def _():
        e0 = e_ref[...]
        bsq_ref[...] = jnp.sum(e0 * e0, axis=1).reshape(1, _K)

    x = x_ref[...]
    a = jnp.sum(x * x, axis=1, keepdims=True)
    b = bsq_ref[...]
    c = jax.lax.dot_general(
        x, e_ref[...], (((1,), (1,)), ((), ())),
        preferred_element_type=jnp.float32)
    d = (a + b) - 2.0 * c
    dmin = jnp.min(d, axis=1, keepdims=True)
    col = jax.lax.broadcasted_iota(jnp.int32, (_BR, _K), 1)
    idx = jnp.min(jnp.where(d == dmin, col, _K), axis=1).astype(jnp.int32)
    idx_ref[0, 0, :] = idx
    onehot = (col == idx[:, None]).astype(jnp.float32)
    enc_ref[...] = onehot
    ones_row = jnp.ones((1, _BR), dtype=jnp.float32)
    pcnt = jax.lax.dot_general(
        ones_row, onehot, (((1,), (0,)), ((), ())),
        preferred_element_type=jnp.float32)
    ploss = jnp.sum(dmin.reshape(2, _BR // 2), axis=0, keepdims=True)

    @pl.when(i == 0)
    def _():
        cnt_acc[...] = pcnt
        loss_acc[...] = ploss

    @pl.when(i > 0)
    def _():
        cnt_acc[...] += pcnt
        loss_acc[...] += ploss

    @pl.when(i == _NB - 1)
    def _():
        cnt_ref[...] = cnt_acc[...]
        loss_ref[...] = loss_acc[...]


_vq_call = pl.pallas_call(
    _vq_body,
    grid=(_NB,),
    in_specs=[
        pl.BlockSpec((_BR, _D), lambda i: (i, 0)),
        pl.BlockSpec((_K, _D), lambda i: (0, 0)),
    ],
    out_specs=[
        pl.BlockSpec((1, 1, _BR), lambda i: (i, 0, 0)),
        pl.BlockSpec((_BR, _K), lambda i: (i, 0)),
        pl.BlockSpec((1, _K), lambda i: (0, 0)),
        pl.BlockSpec((1, _BR // 2), lambda i: (0, 0)),
    ],
    out_shape=[
        jax.ShapeDtypeStruct((_NB, 1, _BR), jnp.int32),
        jax.ShapeDtypeStruct((_B, _K), jnp.float32),
        jax.ShapeDtypeStruct((1, _K), jnp.float32),
        jax.ShapeDtypeStruct((1, _BR // 2), jnp.float32),
    ],
    scratch_shapes=[
        pltpu.VMEM((1, _K), jnp.float32),
        pltpu.VMEM((1, _K), jnp.float32),
        pltpu.VMEM((1, _BR // 2), jnp.float32),
    ],
)

_SC_INFO = plsc.get_sparse_core_info()
_NC = _SC_INFO.num_cores
_NS = _SC_INFO.num_subcores
_NW = _NC * _NS
_BPW = _B // _NW
_CH = 128
_NCH = _BPW // _CH


@functools.partial(
    pl.kernel,
    mesh=plsc.VectorSubcoreMesh(core_axis_name="c", subcore_axis_name="s"),
    out_type=jax.ShapeDtypeStruct((_B, _D), jnp.float32),
    scratch_types=[
        pltpu.VMEM((_CH,), jnp.int32),
        pltpu.VMEM((_CH, _D), jnp.float32),
        pltpu.SemaphoreType.DMA,
    ],
)
def _sc_gather(idx_hbm, table_hbm, out_hbm, idx_v, rows_v, sem):
    wid = lax.axis_index("s") * _NC + lax.axis_index("c")
    base = wid * _BPW
    for ci in range(_NCH):
        off = base + ci * _CH
        pltpu.sync_copy(idx_hbm.at[pl.ds(off, _CH)], idx_v)
        pltpu.async_copy(table_hbm.at[idx_v], rows_v, sem).wait()
        pltpu.sync_copy(rows_v, out_hbm.at[pl.ds(off, _CH)])


def kernel(inputs, label, embedding_weight):
    idx3, enc, cnt, losspart = _vq_call(inputs, embedding_weight)
    q = _sc_gather(idx3.reshape(_B), embedding_weight)
    a = jnp.sum(losspart) / (_B * _D)
    loss = a + _CC * a
    p = cnt[0] / _B
    perplexity = jnp.exp(-jnp.sum(p * jnp.log(p + 1e-10)))
    return (loss, q, perplexity, enc)


# speedup vs baseline: 1.5828x; 1.0046x over previous
import functools

import jax
import jax.numpy as jnp
from jax import lax
from jax.experimental import pallas as pl
from jax.experimental.pallas import tpu as pltpu
from jax.experimental.pallas import tpu_sc as plsc

_K = 8192
_D = 256
_B = 32768
_BR = 256
_NB = _B // _BR
_CC = 0.25


def _vq_body(x_ref, e_ref, idx_ref, enc_ref, cnt_ref, loss_ref, bsq_ref,
             cnt_acc, loss_acc):
    i = pl.program_id(0)

    @pl.when(i == 0)
    def _():
        e0 = e_ref[...]
        bsq_ref[...] = jnp.sum(e0 * e0, axis=1).reshape(1, _K)

    x = x_ref[...]
    a = jnp.sum(x * x, axis=1, keepdims=True)
    b = bsq_ref[...]
    c = jax.lax.dot_general(
        x, e_ref[...], (((1,), (1,)), ((), ())),
        preferred_element_type=jnp.float32)
    d = (a + b) - 2.0 * c
    dmin = jnp.min(d, axis=1, keepdims=True)
    col = jax.lax.broadcasted_iota(jnp.int32, (_BR, _K), 1)
    idx = jnp.min(jnp.where(d == dmin, col, _K), axis=1).astype(jnp.int32)
    idx_ref[0, 0, :] = idx
    onehot = (col == idx[:, None]).astype(jnp.float32)
    enc_ref[...] = onehot
    ones_row = jnp.ones((1, _BR), dtype=jnp.float32)
    pcnt = jax.lax.dot_general(
        ones_row, onehot, (((1,), (0,)), ((), ())),
        preferred_element_type=jnp.float32)
    ploss = jnp.sum(dmin.reshape(2, _BR // 2), axis=0, keepdims=True)

    @pl.when(i == 0)
    def _():
        cnt_acc[...] = pcnt
        loss_acc[...] = ploss

    @pl.when(i > 0)
    def _():
        cnt_acc[...] += pcnt
        loss_acc[...] += ploss

    @pl.when(i == _NB - 1)
    def _():
        cnt_ref[...] = cnt_acc[...]
        loss_ref[...] = loss_acc[...]


_vq_call = pl.pallas_call(
    _vq_body,
    grid=(_NB,),
    in_specs=[
        pl.BlockSpec((_BR, _D), lambda i: (i, 0)),
        pl.BlockSpec((_K, _D), lambda i: (0, 0)),
    ],
    out_specs=[
        pl.BlockSpec((1, 1, _BR), lambda i: (i, 0, 0)),
        pl.BlockSpec((_BR, _K), lambda i: (i, 0)),
        pl.BlockSpec((1, _K), lambda i: (0, 0)),
        pl.BlockSpec((1, _BR // 2), lambda i: (0, 0)),
    ],
    out_shape=[
        jax.ShapeDtypeStruct((_NB, 1, _BR), jnp.int32),
        jax.ShapeDtypeStruct((_B, _K), jnp.float32),
        jax.ShapeDtypeStruct((1, _K), jnp.float32),
        jax.ShapeDtypeStruct((1, _BR // 2), jnp.float32),
    ],
    scratch_shapes=[
        pltpu.VMEM((1, _K), jnp.float32),
        pltpu.VMEM((1, _K), jnp.float32),
        pltpu.VMEM((1, _BR // 2), jnp.float32),
    ],
)

_SC_INFO = plsc.get_sparse_core_info()
_NC = _SC_INFO.num_cores
_NS = _SC_INFO.num_subcores
_NW = _NC * _NS
_BPW = _B // _NW
_CH = 128
_NCH = _BPW // _CH


@functools.partial(
    pl.kernel,
    mesh=plsc.VectorSubcoreMesh(core_axis_name="c", subcore_axis_name="s"),
    out_type=jax.ShapeDtypeStruct((_B, _D), jnp.float32),
    scratch_types=[
        pltpu.VMEM((_BPW,), jnp.int32),
        pltpu.VMEM((_CH, _D), jnp.float32),
        pltpu.VMEM((_CH, _D), jnp.float32),
        pltpu.SemaphoreType.DMA,
        pltpu.SemaphoreType.DMA,
        pltpu.SemaphoreType.DMA,
        pltpu.SemaphoreType.DMA,
    ],
)
def _sc_gather(idx_hbm, table_hbm, out_hbm, idx_v, rows0, rows1, g0, g1,
               w0, w1):
    wid = lax.axis_index("s") * _NC + lax.axis_index("c")
    base = wid * _BPW
    pltpu.sync_copy(idx_hbm.at[pl.ds(base, _BPW)], idx_v)
    rows = (rows0, rows1)
    gsem = (g0, g1)
    wsem = (w0, w1)
    pltpu.async_copy(table_hbm.at[idx_v.at[pl.ds(0, _CH)]], rows0, g0)
    for ci in range(_NCH):
        p = ci % 2
        pltpu.make_async_copy(table_hbm.at[idx_v.at[pl.ds(ci * _CH, _CH)]],
                              rows[p], gsem[p]).wait()
        if ci + 1 < _NCH:
            if ci >= 1:
                pltpu.make_async_copy(
                    rows[(ci + 1) % 2],
                    out_hbm.at[pl.ds(base + (ci - 1) * _CH, _CH)],
                    wsem[(ci + 1) % 2]).wait()
            pltpu.async_copy(
                table_hbm.at[idx_v.at[pl.ds((ci + 1) * _CH, _CH)]],
                rows[(ci + 1) % 2], gsem[(ci + 1) % 2])
        pltpu.async_copy(rows[p], out_hbm.at[pl.ds(base + ci * _CH, _CH)],
                         wsem[p])
    pltpu.make_async_copy(rows[(_NCH - 2) % 2],
                          out_hbm.at[pl.ds(base + (_NCH - 2) * _CH, _CH)],
                          wsem[(_NCH - 2) % 2]).wait()
    pltpu.make_async_copy(rows[(_NCH - 1) % 2],
                          out_hbm.at[pl.ds(base + (_NCH - 1) * _CH, _CH)],
                          wsem[(_NCH - 1) % 2]).wait()


def kernel(inputs, label, embedding_weight):
    idx3, enc, cnt, losspart = _vq_call(inputs, embedding_weight)
    q = _sc_gather(idx3.reshape(_B), embedding_weight)
    a = jnp.sum(losspart) / (_B * _D)
    loss = a + _CC * a
    p = cnt[0] / _B
    perplexity = jnp.exp(-jnp.sum(p * jnp.log(p + 1e-10)))
    return (loss, q, perplexity, enc)
